# baseline (speedup 1.0000x reference)
import functools

import jax
import jax.numpy as jnp
import numpy as np
from jax import lax
from jax.experimental import pallas as pl
from jax.experimental.pallas import tpu as pltpu
from jax.experimental.pallas import tpu_sc as plsc

_BATCH = 4096
_F = 26
_V = 100000
_E = 16
_P = _F * (_F - 1) // 2
_XW = _F * _E
_PE = _P * _E
_H1 = 400

_row = []
_col = []
for _i in range(_F - 1):
    for _j in range(_i + 1, _F):
        _row.append(_i)
        _col.append(_j)
_ROWF = np.array(_row, dtype=np.int32)
_COLF = np.array(_col, dtype=np.int32)


_NC = 2
_NS = 16
_NW = _NC * _NS
_TOT = _BATCH * _F
_PER_W = _TOT // _NW
_CHUNK = 128
_NCHUNK = _PER_W // _CHUNK
_G = 8


def _sc_gather_body(table_hbm, gidx_hbm, ext_hbm, out_hbm,
                    gidx_v, ext_v, grp_a, grp_b, out_v, sem):
    wid = lax.axis_index("s") * _NC + lax.axis_index("c")
    base = wid * _PER_W
    pltpu.sync_copy(gidx_hbm.at[wid], gidx_v)
    pltpu.sync_copy(ext_hbm.at[wid], ext_v)
    lanes = lax.iota(jnp.int32, 16)
    bufs = (grp_a, grp_b)

    def extract(j, grp):
        def body(k, carry):
            ev = ext_v[j, pl.ds(k * 16, 16)]
            rows = k * 16 + lanes
            oaddr = (j * _CHUNK + rows) * _E
            for e in range(_E):
                vals = plsc.load_gather(grp, [rows, ev + e])
                plsc.store_scatter(out_v, [oaddr + e], vals)
            return carry

        lax.fori_loop(0, _CHUNK // 16, body, 0)

    cps = [pltpu.async_copy(table_hbm.at[gidx_v.at[0]], grp_a, sem)]
    for j in range(_NCHUNK):
        cps[j].wait()
        if j + 1 < _NCHUNK:
            cps.append(
                pltpu.async_copy(
                    table_hbm.at[gidx_v.at[j + 1]], bufs[(j + 1) % 2], sem
                )
            )
        extract(j, bufs[j % 2])
    pltpu.sync_copy(out_v, out_hbm.at[pl.ds(base * _E, _PER_W * _E)])


@functools.lru_cache(maxsize=1)
def _get_sc_gather():
    return functools.partial(
        pl.kernel,
        mesh=plsc.VectorSubcoreMesh(core_axis_name="c", subcore_axis_name="s"),
        out_type=jax.ShapeDtypeStruct((_TOT * _E,), jnp.float32),
        scratch_types=[
            pltpu.VMEM((_NCHUNK, _CHUNK), jnp.int32),
            pltpu.VMEM((_NCHUNK, _CHUNK), jnp.int32),
            pltpu.VMEM((_CHUNK, _G * _E), jnp.float32),
            pltpu.VMEM((_CHUNK, _G * _E), jnp.float32),
            pltpu.VMEM((_PER_W * _E,), jnp.float32),
            pltpu.SemaphoreType.DMA,
        ],
        compiler_params=pltpu.CompilerParams(needs_layout_passes=False),
    )(_sc_gather_body)


_BT = 256


def _tc_body(xw_ref, m1_ref, w0a_ref, w0b_ref, b0_ref, w1_ref, b1_ref, out_ref):
    xw = xw_ref[...]
    t = jnp.dot(xw, m1_ref[...], preferred_element_type=jnp.float32)
    parts = []
    for f in range(_F - 1):
        n = _F - 1 - f
        xf = xw[:, f * _E:(f + 1) * _E]
        parts.append(jnp.concatenate([xf] * n, axis=1))
    v = t * jnp.concatenate(parts, axis=1)
    kp = jnp.sum(v.reshape(_BT, _P, _E), axis=-1)
    z0 = (
        jnp.dot(xw, w0a_ref[...], preferred_element_type=jnp.float32)
        + jnp.dot(kp, w0b_ref[...], preferred_element_type=jnp.float32)
        + b0_ref[...]
    )
    h = jnp.maximum(z0, 0.0)
    z1 = jnp.dot(h, w1_ref[...], preferred_element_type=jnp.float32) + b1_ref[...]
    out_ref[...] = 1.0 / (1.0 + jnp.exp(-z1))


def _tc_call(xw, m1, w0a, w0b, b0, w1, b1):
    grid = (_BATCH // _BT,)
    return pl.pallas_call(
        _tc_body,
        grid=grid,
        in_specs=[
            pl.BlockSpec((_BT, _XW), lambda i: (i, 0)),
            pl.BlockSpec((_XW, _PE), lambda i: (0, 0)),
            pl.BlockSpec((_XW, _H1), lambda i: (0, 0)),
            pl.BlockSpec((_P, _H1), lambda i: (0, 0)),
            pl.BlockSpec((1, _H1), lambda i: (0, 0)),
            pl.BlockSpec((_H1, 1), lambda i: (0, 0)),
            pl.BlockSpec((1, 1), lambda i: (0, 0)),
        ],
        out_specs=pl.BlockSpec((_BT, 1), lambda i: (i, 0)),
        out_shape=jax.ShapeDtypeStruct((_BATCH, 1), jnp.float32),
    )(xw, m1, w0a, w0b, b0, w1, b1)


def kernel(indices, embed, kernel, w0, b0, w1, b1):
    table = embed.reshape(_F * _V // _G, _G * _E)
    glob = (indices + (jnp.arange(_F, dtype=jnp.int32) * _V)[None, :]).reshape(-1)
    gidx = (glob // _G).reshape(_NW, _NCHUNK, _CHUNK)
    ext = ((glob % _G) * _E).reshape(_NW, _NCHUNK, _CHUNK)
    rows = _get_sc_gather()(table, gidx, ext)
    xw = rows.reshape(_BATCH, _XW)

    onehot = (
        _COLF[None, :] == jnp.arange(_F, dtype=jnp.int32)[:, None]
    ).astype(jnp.float32)
    m1 = (onehot[:, None, :, None] * kernel[None, :, :, :]).reshape(_XW, _PE)
    w0a = w0[:_XW]
    w0b = w0[_XW:]
    b0r = b0.reshape(1, _H1)
    b1r = b1.reshape(1, 1)

    out = _tc_call(xw, m1, w0a, w0b, b0r, w1, b1r)
    return out.reshape(_BATCH)


# speedup vs baseline: 1.0043x; 1.0043x over previous
import functools

import jax
import jax.numpy as jnp
import numpy as np
from jax import lax
from jax.experimental import pallas as pl
from jax.experimental.pallas import tpu as pltpu
from jax.experimental.pallas import tpu_sc as plsc

_BATCH = 4096
_F = 26
_V = 100000
_E = 16
_P = _F * (_F - 1) // 2
_XW = _F * _E
_PE = _P * _E
_H1 = 400

_row = []
_col = []
for _i in range(_F - 1):
    for _j in range(_i + 1, _F):
        _row.append(_i)
        _col.append(_j)
_ROWF = np.array(_row, dtype=np.int32)
_COLF = np.array(_col, dtype=np.int32)


_NC = 2
_NS = 16
_NW = _NC * _NS
_TOT = _BATCH * _F
_PER_W = _TOT // _NW
_CHUNK = 128
_NCHUNK = _PER_W // _CHUNK
_G = 8


def _sc_gather_body(table_hbm, gidx_hbm, ext_hbm, out_hbm,
                    gidx_v, ext_v, grp_a, grp_b, out_v, sem):
    wid = lax.axis_index("s") * _NC + lax.axis_index("c")
    base = wid * _PER_W
    pltpu.sync_copy(gidx_hbm.at[wid], gidx_v)
    pltpu.sync_copy(ext_hbm.at[wid], ext_v)
    lanes = lax.iota(jnp.int32, 16)
    bufs = (grp_a, grp_b)

    def extract(j, grp):
        def body(k, carry):
            ev = ext_v[j, pl.ds(k * 16, 16)]
            rows = k * 16 + lanes
            oaddr = (j * _CHUNK + rows) * _E
            for e in range(_E):
                vals = plsc.load_gather(grp, [rows, ev + e])
                plsc.store_scatter(out_v, [oaddr + e], vals)
            return carry

        lax.fori_loop(0, _CHUNK // 16, body, 0)

    cps = [pltpu.async_copy(table_hbm.at[gidx_v.at[0]], grp_a, sem)]
    for j in range(_NCHUNK):
        cps[j].wait()
        if j + 1 < _NCHUNK:
            cps.append(
                pltpu.async_copy(
                    table_hbm.at[gidx_v.at[j + 1]], bufs[(j + 1) % 2], sem
                )
            )
        extract(j, bufs[j % 2])
    pltpu.sync_copy(out_v, out_hbm.at[pl.ds(base * _E, _PER_W * _E)])


@functools.lru_cache(maxsize=1)
def _get_sc_gather():
    return functools.partial(
        pl.kernel,
        mesh=plsc.VectorSubcoreMesh(core_axis_name="c", subcore_axis_name="s"),
        out_type=jax.ShapeDtypeStruct((_TOT * _E,), jnp.float32),
        scratch_types=[
            pltpu.VMEM((_NCHUNK, _CHUNK), jnp.int32),
            pltpu.VMEM((_NCHUNK, _CHUNK), jnp.int32),
            pltpu.VMEM((_CHUNK, _G * _E), jnp.float32),
            pltpu.VMEM((_CHUNK, _G * _E), jnp.float32),
            pltpu.VMEM((_PER_W * _E,), jnp.float32),
            pltpu.SemaphoreType.DMA,
        ],
        compiler_params=pltpu.CompilerParams(needs_layout_passes=False),
    )(_sc_gather_body)


_BT = 256

_RUN_OFF = np.concatenate([[0], np.cumsum(_F - 1 - np.arange(_F - 1))]).astype(np.int32)
_BLK_COLS = [
    int((_RUN_OFF[8] - _RUN_OFF[0]) * _E),
    int((_RUN_OFF[16] - _RUN_OFF[8]) * _E),
    int((_RUN_OFF[25] - _RUN_OFF[16]) * _E),
]


def _tc_body(xw_ref, wa_ref, wb_ref, wc_ref, w0a_ref, w0b_ref, b0_ref,
             w1_ref, b1_ref, out_ref):
    xw = xw_ref[...]
    ta = jnp.dot(xw[:, 0:128], wa_ref[...], preferred_element_type=jnp.float32)
    tb = jnp.dot(xw[:, 128:256], wb_ref[...], preferred_element_type=jnp.float32)
    tc = jnp.dot(xw[:, 256:400], wc_ref[...], preferred_element_type=jnp.float32)
    t = jnp.concatenate([ta, tb, tc], axis=1)
    parts = []
    for f in range(_F - 1):
        o = int(_RUN_OFF[f]) * _E
        n = (_F - 1 - f) * _E
        parts.append(t[:, o:o + n] * xw[:, (f + 1) * _E:])
    v = jnp.concatenate(parts, axis=1)
    kp = jnp.sum(v.reshape(_BT, _P, _E), axis=-1)
    z0 = (
        jnp.dot(xw, w0a_ref[...], preferred_element_type=jnp.float32)
        + jnp.dot(kp, w0b_ref[...], preferred_element_type=jnp.float32)
        + b0_ref[...]
    )
    h = jnp.maximum(z0, 0.0)
    z1 = jnp.dot(h, w1_ref[...], preferred_element_type=jnp.float32) + b1_ref[...]
    out_ref[...] = 1.0 / (1.0 + jnp.exp(-z1))


def _tc_call(xw, wa, wb, wc, w0a, w0b, b0, w1, b1):
    grid = (_BATCH // _BT,)
    return pl.pallas_call(
        _tc_body,
        grid=grid,
        in_specs=[
            pl.BlockSpec((_BT, _XW), lambda i: (i, 0)),
            pl.BlockSpec((128, _BLK_COLS[0]), lambda i: (0, 0)),
            pl.BlockSpec((128, _BLK_COLS[1]), lambda i: (0, 0)),
            pl.BlockSpec((144, _BLK_COLS[2]), lambda i: (0, 0)),
            pl.BlockSpec((_XW, _H1), lambda i: (0, 0)),
            pl.BlockSpec((_P, _H1), lambda i: (0, 0)),
            pl.BlockSpec((1, _H1), lambda i: (0, 0)),
            pl.BlockSpec((_H1, 1), lambda i: (0, 0)),
            pl.BlockSpec((1, 1), lambda i: (0, 0)),
        ],
        out_specs=pl.BlockSpec((_BT, 1), lambda i: (i, 0)),
        out_shape=jax.ShapeDtypeStruct((_BATCH, 1), jnp.float32),
    )(xw, wa, wb, wc, w0a, w0b, b0, w1, b1)


def kernel(indices, embed, kernel, w0, b0, w1, b1):
    table = embed.reshape(_F * _V // _G, _G * _E)
    glob = (indices + (jnp.arange(_F, dtype=jnp.int32) * _V)[None, :]).reshape(-1)
    gidx = (glob // _G).reshape(_NW, _NCHUNK, _CHUNK)
    ext = ((glob % _G) * _E).reshape(_NW, _NCHUNK, _CHUNK)
    rows = _get_sc_gather()(table, gidx, ext)
    xw = rows.reshape(_BATCH, _XW)

    onehot = (
        _ROWF[None, :] == jnp.arange(_F, dtype=jnp.int32)[:, None]
    ).astype(jnp.float32)
    ktr = jnp.transpose(kernel, (2, 1, 0))
    wfull = (onehot[:, None, :, None] * ktr[None, :, :, :]).reshape(_XW, _PE)
    c0, c1 = _BLK_COLS[0], _BLK_COLS[1]
    wa = wfull[0:128, 0:c0]
    wb = wfull[128:256, c0:c0 + c1]
    wc = wfull[256:400, c0 + c1:]
    w0a = w0[:_XW]
    w0b = w0[_XW:]
    b0r = b0.reshape(1, _H1)
    b1r = b1.reshape(1, 1)

    out = _tc_call(xw, wa, wb, wc, w0a, w0b, b0r, w1, b1r)
    return out.reshape(_BATCH)


# speedup vs baseline: 1.0057x; 1.0014x over previous
import functools

import jax
import jax.numpy as jnp
import numpy as np
from jax import lax
from jax.experimental import pallas as pl
from jax.experimental.pallas import tpu as pltpu
from jax.experimental.pallas import tpu_sc as plsc

_BATCH = 4096
_F = 26
_V = 100000
_E = 16
_P = _F * (_F - 1) // 2
_XW = _F * _E
_PE = _P * _E
_H1 = 400

_row = []
_col = []
for _i in range(_F - 1):
    for _j in range(_i + 1, _F):
        _row.append(_i)
        _col.append(_j)
_ROWF = np.array(_row, dtype=np.int32)
_COLF = np.array(_col, dtype=np.int32)


_NC = 2
_NS = 16
_NW = _NC * _NS
_TOT = _BATCH * _F
_PER_W = _TOT // _NW
_CHUNK = 128
_NCHUNK = _PER_W // _CHUNK
_G = 8


def _sc_gather_body(table_hbm, gidx_hbm, ext_hbm, out_hbm,
                    gidx_v, ext_v, grp_a, grp_b, out_v, sem):
    wid = lax.axis_index("s") * _NC + lax.axis_index("c")
    base = wid * _PER_W
    pltpu.sync_copy(gidx_hbm.at[wid], gidx_v)
    pltpu.sync_copy(ext_hbm.at[wid], ext_v)
    lanes = lax.iota(jnp.int32, 16)
    bufs = (grp_a, grp_b)

    def extract(j, grp):
        def body(k, carry):
            jv = ext_v[j, pl.ds(k * 16, 16)]
            rows = k * 16 + lanes
            oaddr = (j * _CHUNK + rows) * _E
            for e in range(_E):
                vals = plsc.load_gather(grp, [rows, jv + e])
                plsc.store_scatter(out_v, [oaddr + e], vals)
            return carry

        lax.fori_loop(0, _CHUNK // 16, body, 0)

    cps = [pltpu.async_copy(table_hbm.at[gidx_v.at[0]], grp_a, sem)]
    for j in range(_NCHUNK):
        cps[j].wait()
        if j + 1 < _NCHUNK:
            cps.append(
                pltpu.async_copy(
                    table_hbm.at[gidx_v.at[j + 1]], bufs[(j + 1) % 2], sem
                )
            )
        extract(j, bufs[j % 2])
    pltpu.sync_copy(out_v, out_hbm.at[pl.ds(base * _E, _PER_W * _E)])


@functools.lru_cache(maxsize=1)
def _get_sc_gather():
    return functools.partial(
        pl.kernel,
        mesh=plsc.VectorSubcoreMesh(core_axis_name="c", subcore_axis_name="s"),
        out_type=jax.ShapeDtypeStruct((_TOT * _E,), jnp.float32),
        scratch_types=[
            pltpu.VMEM((_NCHUNK, _CHUNK), jnp.int32),
            pltpu.VMEM((_NCHUNK, _CHUNK), jnp.int32),
            pltpu.VMEM((_CHUNK, _G * _E), jnp.float32),
            pltpu.VMEM((_CHUNK, _G * _E), jnp.float32),
            pltpu.VMEM((_PER_W * _E,), jnp.float32),
            pltpu.SemaphoreType.DMA,
        ],
        compiler_params=pltpu.CompilerParams(needs_layout_passes=False),
    )(_sc_gather_body)


_BT = 256

_RUN_OFF = np.concatenate([[0], np.cumsum(_F - 1 - np.arange(_F - 1))]).astype(np.int32)
_BLK_COLS = [
    int((_RUN_OFF[8] - _RUN_OFF[0]) * _E),
    int((_RUN_OFF[16] - _RUN_OFF[8]) * _E),
    int((_RUN_OFF[25] - _RUN_OFF[16]) * _E),
]


def _tc_body(xw_ref, wa_ref, wb_ref, wc_ref, w0a_ref, w0b_ref, b0_ref,
             w1_ref, b1_ref, out_ref):
    xw = xw_ref[...]
    ta = jnp.dot(xw[:, 0:128], wa_ref[...], preferred_element_type=jnp.float32)
    tb = jnp.dot(xw[:, 128:256], wb_ref[...], preferred_element_type=jnp.float32)
    tc = jnp.dot(xw[:, 256:400], wc_ref[...], preferred_element_type=jnp.float32)
    t = jnp.concatenate([ta, tb, tc], axis=1)
    parts = []
    for f in range(_F - 1):
        o = int(_RUN_OFF[f]) * _E
        n = (_F - 1 - f) * _E
        parts.append(t[:, o:o + n] * xw[:, (f + 1) * _E:])
    v = jnp.concatenate(parts, axis=1)
    kp = jnp.sum(v.reshape(_BT, _P, _E), axis=-1)
    z0 = (
        jnp.dot(xw, w0a_ref[...], preferred_element_type=jnp.float32)
        + jnp.dot(kp, w0b_ref[...], preferred_element_type=jnp.float32)
        + b0_ref[...]
    )
    h = jnp.maximum(z0, 0.0)
    z1 = jnp.dot(h, w1_ref[...], preferred_element_type=jnp.float32) + b1_ref[...]
    out_ref[...] = 1.0 / (1.0 + jnp.exp(-z1))


def _tc_call(xw, wa, wb, wc, w0a, w0b, b0, w1, b1):
    grid = (_BATCH // _BT,)
    return pl.pallas_call(
        _tc_body,
        grid=grid,
        in_specs=[
            pl.BlockSpec((_BT, _XW), lambda i: (i, 0)),
            pl.BlockSpec((128, _BLK_COLS[0]), lambda i: (0, 0)),
            pl.BlockSpec((128, _BLK_COLS[1]), lambda i: (0, 0)),
            pl.BlockSpec((144, _BLK_COLS[2]), lambda i: (0, 0)),
            pl.BlockSpec((_XW, _H1), lambda i: (0, 0)),
            pl.BlockSpec((_P, _H1), lambda i: (0, 0)),
            pl.BlockSpec((1, _H1), lambda i: (0, 0)),
            pl.BlockSpec((_H1, 1), lambda i: (0, 0)),
            pl.BlockSpec((1, 1), lambda i: (0, 0)),
        ],
        out_specs=pl.BlockSpec((_BT, 1), lambda i: (i, 0)),
        out_shape=jax.ShapeDtypeStruct((_BATCH, 1), jnp.float32),
    )(xw, wa, wb, wc, w0a, w0b, b0, w1, b1)


def kernel(indices, embed, kernel, w0, b0, w1, b1):
    table = embed.reshape(_F * _V // _G, _G * _E)
    glob = (indices + (jnp.arange(_F, dtype=jnp.int32) * _V)[None, :]).reshape(-1)
    gidx = (glob // _G).reshape(_NW, _NCHUNK, _CHUNK)
    ext = ((glob % _G) * _E).reshape(_NW, _NCHUNK, _CHUNK)
    rows = _get_sc_gather()(table, gidx, ext)
    xw = rows.reshape(_BATCH, _XW)

    onehot = (
        _ROWF[None, :] == jnp.arange(_F, dtype=jnp.int32)[:, None]
    ).astype(jnp.float32)
    ktr = jnp.transpose(kernel, (2, 1, 0))
    wfull = (onehot[:, None, :, None] * ktr[None, :, :, :]).reshape(_XW, _PE)
    c0, c1 = _BLK_COLS[0], _BLK_COLS[1]
    wa = wfull[0:128, 0:c0]
    wb = wfull[128:256, c0:c0 + c1]
    wc = wfull[256:400, c0 + c1:]
    w0a = w0[:_XW]
    w0b = w0[_XW:]
    b0r = b0.reshape(1, _H1)
    b1r = b1.reshape(1, 1)

    out = _tc_call(xw, wa, wb, wc, w0a, w0b, b0r, w1, b1r)
    return out.reshape(_BATCH)
